# baseline (speedup 1.0000x reference)
import jax
import jax.numpy as jnp
from jax.experimental import pallas as pl

MAX_POS_ = 2048
HIDDEN_ = 768
BATCH_ = 4

BP = 256


def _add_body(hid_ref, pos_ref, out_ref):
    out_ref[...] = hid_ref[...] + pos_ref[...]


def kernel(hidden_states, pos_table):
    grid = (MAX_POS_ // BP,)
    return pl.pallas_call(
        _add_body,
        grid=grid,
        in_specs=[
            pl.BlockSpec((BATCH_, BP, HIDDEN_), lambda i: (0, i, 0)),
            pl.BlockSpec((BP, HIDDEN_), lambda i: (i, 0)),
        ],
        out_specs=pl.BlockSpec((BATCH_, BP, HIDDEN_), lambda i: (0, i, 0)),
        out_shape=jax.ShapeDtypeStruct((BATCH_, MAX_POS_, HIDDEN_), jnp.float32),
    )(hidden_states, pos_table)


# speedup vs baseline: 1.1550x; 1.1550x over previous
import jax
import jax.numpy as jnp
from jax.experimental import pallas as pl

MAX_POS_ = 2048
HIDDEN_ = 768
BATCH_ = 4


def _add_body(hid_ref, pos_ref, out_ref):
    out_ref[...] = hid_ref[...] + pos_ref[...]


def kernel(hidden_states, pos_table):
    grid = (2,)
    return pl.pallas_call(
        _add_body,
        grid=grid,
        in_specs=[
            pl.BlockSpec((2, MAX_POS_, HIDDEN_), lambda i: (i, 0, 0)),
            pl.BlockSpec((MAX_POS_, HIDDEN_), lambda i: (0, 0)),
        ],
        out_specs=pl.BlockSpec((2, MAX_POS_, HIDDEN_), lambda i: (i, 0, 0)),
        out_shape=jax.ShapeDtypeStruct((BATCH_, MAX_POS_, HIDDEN_), jnp.float32),
    )(hidden_states, pos_table)
